# Initial kernel scaffold; baseline (speedup 1.0000x reference)
import jax
import jax.numpy as jnp
from jax.experimental import pallas as pl


def kernel(pos, batch, params):
    raise NotImplementedError("write your pallas kernel here")


# speedup vs baseline: 2.7764x; 2.7764x over previous
import jax
import jax.numpy as jnp
from jax.experimental import pallas as pl
from jax.experimental.pallas import tpu as pltpu

_BIG = 1e10
_F32 = jnp.float32


def _fps_call(pos0T, P, B, n1, n2, n3):
    def kern(p0_ref, o1_ref, o2_ref, o3_ref):
        def run_level(X, Y, Z, n, o_ref, Pc):
            lane = jax.lax.broadcasted_iota(jnp.int32, X.shape, 1)
            olane = jax.lax.broadcasted_iota(jnp.int32, (B, n), 1)
            sel0 = (olane == 0).astype(_F32)
            cx = X[:, 0:1]
            cy = Y[:, 0:1]
            cz = Z[:, 0:1]
            ox = cx * sel0
            oy = cy * sel0
            oz = cz * sel0
            minds = jnp.full(X.shape, _BIG, _F32)

            def body(i, st):
                minds, ox, oy, oz, cx, cy, cz = st
                d = (X - cx) ** 2 + (Y - cy) ** 2 + (Z - cz) ** 2
                minds = jnp.minimum(minds, d)
                mx = jnp.max(minds, axis=1, keepdims=True)
                idx = jnp.min(
                    jnp.where(minds == mx, lane, Pc), axis=1, keepdims=True)
                oh = (lane == idx).astype(_F32)
                nx = jnp.sum(oh * X, axis=1, keepdims=True)
                ny = jnp.sum(oh * Y, axis=1, keepdims=True)
                nz = jnp.sum(oh * Z, axis=1, keepdims=True)
                sel = (olane == i).astype(_F32)
                return (minds, ox + nx * sel, oy + ny * sel, oz + nz * sel,
                        nx, ny, nz)

            st = jax.lax.fori_loop(
                1, n, body, (minds, ox, oy, oz, cx, cy, cz))
            ox, oy, oz = st[1], st[2], st[3]
            o_ref[0] = ox
            o_ref[1] = oy
            o_ref[2] = oz
            return ox, oy, oz

        X, Y, Z = p0_ref[0], p0_ref[1], p0_ref[2]
        x1, y1, z1 = run_level(X, Y, Z, n1, o1_ref, P)
        x2, y2, z2 = run_level(x1, y1, z1, n2, o2_ref, n1)
        run_level(x2, y2, z2, n3, o3_ref, n2)

    return pl.pallas_call(
        kern,
        out_shape=[jax.ShapeDtypeStruct((3, B, n), _F32)
                   for n in (n1, n2, n3)],
    )(pos0T)


def _sa_select(pc, pT, src, r, K, MB):
    B, M, _ = pc.shape
    P = pT.shape[2]
    Cf = src.shape[2]
    r2 = r * r
    pos0 = Cf - 8

    def kern(pc_ref, pT_ref, src_ref, feat_ref, val_ref, d2_scr):
        lane = jax.lax.broadcasted_iota(jnp.int32, (MB, P), 1)
        flane = jax.lax.broadcasted_iota(jnp.int32, (MB, Cf), 1)
        d2 = jnp.zeros((MB, P), _F32)
        pcm = jnp.zeros((MB, Cf), _F32)
        for a in range(3):
            pca = pc_ref[0, :, a:a + 1]
            pja = pT_ref[0, a:a + 1, :]
            d2 = d2 + (pca - pja) ** 2
            pcm = pcm + pca * (flane == (pos0 + a)).astype(_F32)
        d2_scr[...] = jnp.where(d2 <= r2, d2, _BIG)
        src_blk = src_ref[0]
        slot = jax.lax.broadcasted_iota(jnp.int32, (MB, K), 1)

        def body(t, vacc):
            d2m = d2_scr[...]
            mval = jnp.min(d2m, axis=1, keepdims=True)
            idx = jnp.min(
                jnp.where(d2m == mval, lane, P), axis=1, keepdims=True)
            oh = (lane == idx).astype(_F32)
            g = jnp.dot(oh, src_blk, preferred_element_type=_F32,
                        precision=jax.lax.Precision.HIGHEST)
            feat_ref[0, t] = g - pcm
            vacc = vacc + ((slot == t) & (mval < 1e9)).astype(_F32)
            d2_scr[...] = jnp.where(oh > 0, _BIG, d2m)
            return vacc

        vacc = jax.lax.fori_loop(0, K, body, jnp.zeros((MB, K), _F32))
        val_ref[0] = vacc

    feat, val = pl.pallas_call(
        kern,
        grid=(B, M // MB),
        in_specs=[
            pl.BlockSpec((1, MB, 3), lambda b, m: (b, m, 0)),
            pl.BlockSpec((1, 8, P), lambda b, m: (b, 0, 0)),
            pl.BlockSpec((1, P, Cf), lambda b, m: (b, 0, 0)),
        ],
        out_specs=[
            pl.BlockSpec((1, K, MB, Cf), lambda b, m: (b, 0, m, 0)),
            pl.BlockSpec((1, MB, K), lambda b, m: (b, m, 0)),
        ],
        out_shape=[
            jax.ShapeDtypeStruct((B, K, M, Cf), _F32),
            jax.ShapeDtypeStruct((B, M, K), _F32),
        ],
        scratch_shapes=[pltpu.VMEM((MB, P), _F32)],
    )(pc, pT, src)
    return feat, val


def _pass_first_kern(z_ref, mk_ref, w_ref, zo_ref, sums_ref):
    i = pl.program_id(0)
    a = z_ref[...]
    h = jnp.dot(a, w_ref[...], preferred_element_type=_F32)
    zo_ref[...] = h
    mk = mk_ref[...]
    hm = h * mk
    p0 = jnp.sum(hm, axis=0, keepdims=True)
    p1 = jnp.sum(hm * h, axis=0, keepdims=True)
    c = jnp.sum(mk)

    @pl.when(i == 0)
    def _():
        sums_ref[...] = jnp.zeros_like(sums_ref)

    s0 = sums_ref[0:1, :]
    c0 = sums_ref[1:2, :]
    y0 = p0 - c0
    t0 = s0 + y0
    sums_ref[1:2, :] = (t0 - s0) - y0
    sums_ref[0:1, :] = t0
    s1 = sums_ref[2:3, :]
    c1 = sums_ref[3:4, :]
    y1 = p1 - c1
    t1 = s1 + y1
    sums_ref[3:4, :] = (t1 - s1) - y1
    sums_ref[2:3, :] = t1
    sums_ref[4:5, :] += c


def _pass_norm_kern(z_ref, mk_ref, w_ref, ss_ref, zo_ref, sums_ref):
    i = pl.program_id(0)
    a = jnp.maximum(
        (z_ref[...] - ss_ref[0:1, :]) / ss_ref[1:2, :] * ss_ref[2:3, :]
        + ss_ref[3:4, :], 0.0)
    h = jnp.dot(a, w_ref[...], preferred_element_type=_F32)
    zo_ref[...] = h
    mk = mk_ref[...]
    hm = h * mk
    p0 = jnp.sum(hm, axis=0, keepdims=True)
    p1 = jnp.sum(hm * h, axis=0, keepdims=True)
    c = jnp.sum(mk)

    @pl.when(i == 0)
    def _():
        sums_ref[...] = jnp.zeros_like(sums_ref)

    s0 = sums_ref[0:1, :]
    c0 = sums_ref[1:2, :]
    y0 = p0 - c0
    t0 = s0 + y0
    sums_ref[1:2, :] = (t0 - s0) - y0
    sums_ref[0:1, :] = t0
    s1 = sums_ref[2:3, :]
    c1 = sums_ref[3:4, :]
    y1 = p1 - c1
    t1 = s1 + y1
    sums_ref[3:4, :] = (t1 - s1) - y1
    sums_ref[2:3, :] = t1
    sums_ref[4:5, :] += c


def _mm_pass(z, mask, W, ss=None):
    N, Cin = z.shape
    Cout = W.shape[1]
    NB = min(1024, N)
    ins = [z, mask, W]
    in_specs = [
        pl.BlockSpec((NB, Cin), lambda i: (i, 0)),
        pl.BlockSpec((NB, 1), lambda i: (i, 0)),
        pl.BlockSpec((Cin, Cout), lambda i: (0, 0)),
    ]
    if ss is None:
        kern = _pass_first_kern
    else:
        kern = _pass_norm_kern
        ins.append(ss)
        in_specs.append(pl.BlockSpec((4, Cin), lambda i: (0, 0)))
    zo, sums = pl.pallas_call(
        kern,
        grid=(N // NB,),
        in_specs=in_specs,
        out_specs=[
            pl.BlockSpec((NB, Cout), lambda i: (i, 0)),
            pl.BlockSpec((8, Cout), lambda i: (0, 0)),
        ],
        out_shape=[
            jax.ShapeDtypeStruct((N, Cout), _F32),
            jax.ShapeDtypeStruct((8, Cout), _F32),
        ],
    )(*ins)
    return zo, sums


def _maxpool(z4, valT, ss, MBp):
    B, K, M, C = z4.shape

    def kern(z_ref, v_ref, ss_ref, o_ref):
        a = jnp.maximum(
            (z_ref[...] - ss_ref[0:1, :].reshape(1, 1, 1, C))
            / ss_ref[1:2, :].reshape(1, 1, 1, C)
            * ss_ref[2:3, :].reshape(1, 1, 1, C)
            + ss_ref[3:4, :].reshape(1, 1, 1, C), 0.0)
        val = v_ref[...].reshape(1, K, MBp, 1)
        am = jnp.where(val > 0, a, -_BIG)
        o_ref[...] = jnp.max(am, axis=1)

    return pl.pallas_call(
        kern,
        grid=(B, M // MBp),
        in_specs=[
            pl.BlockSpec((1, K, MBp, C), lambda b, m: (b, 0, m, 0)),
            pl.BlockSpec((1, K, MBp), lambda b, m: (b, 0, m)),
            pl.BlockSpec((4, C), lambda b, m: (0, 0)),
        ],
        out_specs=pl.BlockSpec((1, MBp, C), lambda b, m: (b, m, 0)),
        out_shape=jax.ShapeDtypeStruct((B, M, C), _F32),
    )(z4, valT, ss)


def _interp(pf, pTc, src, ss, MB):
    B, Mf, _ = pf.shape
    Mc = pTc.shape[2]
    C = src.shape[2]

    def kern(pf_ref, pT_ref, src_ref, ss_ref, o_ref):
        lane = jax.lax.broadcasted_iota(jnp.int32, (MB, Mc), 1)
        d2 = jnp.zeros((MB, Mc), _F32)
        for a in range(3):
            pca = pf_ref[0, :, a:a + 1]
            pja = pT_ref[0, a:a + 1, :]
            d2 = d2 + (pca - pja) ** 2
        src_blk = src_ref[0]
        mm = ss_ref[0:1, :]
        tt = ss_ref[1:2, :]
        gg = ss_ref[2:3, :]
        bb = ss_ref[3:4, :]
        acc = jnp.zeros((MB, C), _F32)
        ws = jnp.zeros((MB, 1), _F32)
        for _ in range(3):
            mval = jnp.min(d2, axis=1, keepdims=True)
            idx = jnp.min(
                jnp.where(d2 == mval, lane, Mc), axis=1, keepdims=True)
            oh = (lane == idx).astype(_F32)
            g = jnp.dot(oh, src_blk, preferred_element_type=_F32,
                        precision=jax.lax.Precision.HIGHEST)
            gn = jnp.maximum((g - mm) / tt * gg + bb, 0.0)
            w = 1.0 / jnp.maximum(mval, 1e-16)
            acc = acc + w * gn
            ws = ws + w
            d2 = jnp.where(oh > 0, _BIG, d2)
        o_ref[0] = acc / ws

    return pl.pallas_call(
        kern,
        grid=(B, Mf // MB),
        in_specs=[
            pl.BlockSpec((1, MB, 3), lambda b, m: (b, m, 0)),
            pl.BlockSpec((1, 8, Mc), lambda b, m: (b, 0, 0)),
            pl.BlockSpec((1, Mc, C), lambda b, m: (b, 0, 0)),
            pl.BlockSpec((4, C), lambda b, m: (0, 0)),
        ],
        out_specs=pl.BlockSpec((1, MB, C), lambda b, m: (b, m, 0)),
        out_shape=jax.ShapeDtypeStruct((B, Mf, C), _F32),
    )(pf, pTc, src, ss)


def _head_final(z, ss, W2, b2):
    N, C = z.shape
    Co = W2.shape[1]
    NB = min(1024, N)

    def kern(z_ref, ss_ref, w_ref, b_ref, o_ref):
        a = jnp.maximum(
            (z_ref[...] - ss_ref[0:1, :]) / ss_ref[1:2, :] * ss_ref[2:3, :]
            + ss_ref[3:4, :], 0.0)
        o_ref[...] = jnp.dot(
            a, w_ref[...], preferred_element_type=_F32) + b_ref[...]

    return pl.pallas_call(
        kern,
        grid=(N // NB,),
        in_specs=[
            pl.BlockSpec((NB, C), lambda i: (i, 0)),
            pl.BlockSpec((4, C), lambda i: (0, 0)),
            pl.BlockSpec((C, Co), lambda i: (0, 0)),
            pl.BlockSpec((1, Co), lambda i: (0, 0)),
        ],
        out_specs=pl.BlockSpec((NB, Co), lambda i: (i, 0)),
        out_shape=jax.ShapeDtypeStruct((N, Co), _F32),
    )(z, ss, W2, b2)


def _sa_stats(z, val, mask4, cnt, g, b):
    B, M, K = val.shape
    h = jnp.transpose(z.reshape(B, K, M, -1), (0, 2, 1, 3))
    m = (h * mask4).sum(axis=(0, 1, 2)) / cnt
    v = (((h - m) ** 2) * mask4).sum(axis=(0, 1, 2)) / cnt
    t = jnp.sqrt(v + 1e-5)
    return jnp.stack([m, t, g, b])


def _sa_mlp(feat, val, layers):
    B, K, M, Cf = feat.shape
    N = B * K * M
    fz = feat.reshape(N, Cf)
    valT = jnp.transpose(val, (0, 2, 1))
    mask = valT.reshape(N, 1)
    mask4 = val[..., None]
    cnt = jnp.maximum(mask4.sum(), 1.0)
    W0 = jnp.concatenate(
        [layers[0]["W"], jnp.zeros((5, layers[0]["W"].shape[1]), _F32)], 0)
    z, _ = _mm_pass(fz, mask, W0)
    ss = _sa_stats(z, val, mask4, cnt, layers[0]["g"], layers[0]["b"])
    z, _ = _mm_pass(z, mask, layers[1]["W"], ss)
    ss = _sa_stats(z, val, mask4, cnt, layers[1]["g"], layers[1]["b"])
    z, _ = _mm_pass(z, mask, layers[2]["W"], ss)
    ss = _sa_stats(z, val, mask4, cnt, layers[2]["g"], layers[2]["b"])
    C = z.shape[1]
    z4 = z.reshape(B, K, M, C)
    MBp = 128 if M % 128 == 0 else M
    return _maxpool(z4, valT, ss, MBp)


def _bn_stats(z, shape3, g, b):
    h = z.reshape(shape3)
    ax = tuple(range(h.ndim - 1))
    m = h.mean(axis=ax)
    v = ((h - m) ** 2).mean(axis=ax)
    t = jnp.sqrt(v + 1e-5)
    return jnp.stack([m, t, g, b])


def _fp_level(pf, pTc, src, ss_src, x_skip, layers, MB):
    B, Mf, _ = pf.shape
    xi = _interp(pf, pTc, src, ss_src, MB)
    h = xi if x_skip is None else jnp.concatenate([xi, x_skip], axis=2)
    N = B * Mf
    h2 = h.reshape(N, h.shape[2])
    ones = jnp.ones((N, 1), _F32)
    z, _ = _mm_pass(h2, ones, layers[0]["W"])
    ss = _bn_stats(z, (B, Mf, -1), layers[0]["g"], layers[0]["b"])
    z, _ = _mm_pass(z, ones, layers[1]["W"], ss)
    ss = _bn_stats(z, (B, Mf, -1), layers[1]["g"], layers[1]["b"])
    return z, ss


def kernel(pos, batch, params):
    B = 8
    P0 = pos.shape[0] // B
    K = 64
    pos0 = pos.reshape(B, P0, 3) + (batch[-1] + 1 - B).astype(pos.dtype)
    n1, n2, n3 = P0 // 2, P0 // 8, P0 // 32

    pos0T = jnp.transpose(pos0, (2, 0, 1))
    o1, o2, o3 = _fps_call(pos0T, P0, B, n1, n2, n3)
    pos1 = jnp.transpose(o1, (1, 2, 0))
    pos2 = jnp.transpose(o2, (1, 2, 0))
    pos3 = jnp.transpose(o3, (1, 2, 0))

    def padT(planes):
        t = jnp.transpose(planes, (1, 0, 2))
        return jnp.concatenate(
            [t, jnp.zeros((B, 5, t.shape[2]), _F32)], axis=1)

    pT0, pT1, pT2, pT3 = padT(pos0T), padT(o1), padT(o2), padT(o3)

    src1 = jnp.concatenate([pos0, jnp.zeros((B, P0, 5), _F32)], axis=2)
    feat1, val1 = _sa_select(pos1, pT0, src1, 0.2, K, min(128, n1))
    x1 = _sa_mlp(feat1, val1, params["sa1"])

    src2 = jnp.concatenate(
        [x1, pos1, jnp.zeros((B, n1, 5), _F32)], axis=2)
    feat2, val2 = _sa_select(pos2, pT1, src2, 0.4, K, min(128, n2))
    x2 = _sa_mlp(feat2, val2, params["sa2"])

    src3 = jnp.concatenate(
        [x2, pos2, jnp.zeros((B, n2, 5), _F32)], axis=2)
    feat3, val3 = _sa_select(pos3, pT2, src3, 0.8, K, min(128, n3))
    x3 = _sa_mlp(feat3, val3, params["sa3"])

    nc = x3.shape[2]
    ident = jnp.stack([jnp.zeros((nc,), _F32), jnp.ones((nc,), _F32),
                       jnp.ones((nc,), _F32), jnp.zeros((nc,), _F32)])
    z_fp3, ss_fp3 = _fp_level(
        pos2, pT3, x3, ident, x2, params["fp3"], min(128, n2))
    z_fp2, ss_fp2 = _fp_level(
        pos1, pT2, z_fp3.reshape(B, n2, -1), ss_fp3, x1, params["fp2"],
        min(128, n1))
    z_fp1, ss_fp1 = _fp_level(
        pos0, pT1, z_fp2.reshape(B, n1, -1), ss_fp2, None, params["fp1"],
        min(128, P0))

    hd = params["head"]
    ones = jnp.ones((B * P0, 1), _F32)
    zh, _ = _mm_pass(z_fp1, ones, hd["W1"], ss_fp1)
    ss_h = _bn_stats(zh, zh.shape, hd["g"], hd["b"])
    return _head_final(zh, ss_h, hd["W2"], hd["b2"].reshape(1, -1))
